# Initial kernel scaffold; baseline (speedup 1.0000x reference)
import jax
import jax.numpy as jnp
from jax.experimental import pallas as pl


def kernel(x, edge_index, edge_attr, u, batch, W1, b1, W2, b2):
    raise NotImplementedError("write your pallas kernel here")


# speedup vs baseline: 10.6546x; 10.6546x over previous
import functools

import jax
import jax.numpy as jnp
from jax import lax
from jax.experimental import pallas as pl
from jax.experimental.pallas import tpu as pltpu
from jax.experimental.pallas import tpu_sc as plsc

G = 16
N_NODES = 10000
N_EDGES = 160000
F_X = 256
F_E = 16
F_U = 256
H = 512

NODE_CHUNK = 80
EDGE_CHUNK = 1280
NUM_CHUNKS = 125
SUB = 128
CNT_W = 8

NC = 2
NS = 16
NW = NC * NS
CHUNKS_PER_TILE = (NUM_CHUNKS + NW - 1) // NW


def _sc_aggregate(x3d, batch_flat, row2d, ea3d, ones, z256, z16, z8):
    mesh = plsc.VectorSubcoreMesh(core_axis_name="c", subcore_axis_name="s")

    @functools.partial(
        pl.kernel,
        out_type=(
            jax.ShapeDtypeStruct((NC, G, F_X), jnp.float32),
            jax.ShapeDtypeStruct((NC, G, F_E), jnp.float32),
            jax.ShapeDtypeStruct((NC, G, CNT_W), jnp.float32),
            jax.ShapeDtypeStruct((NC, G, CNT_W), jnp.float32),
        ),
        mesh=mesh,
        compiler_params=pltpu.CompilerParams(
            needs_layout_passes=False, use_tc_tiling_on_sc=False),
        scratch_types=[
            pltpu.VMEM((N_NODES,), jnp.int32),
            pltpu.VMEM((NODE_CHUNK, F_X), jnp.float32),
            pltpu.VMEM((NODE_CHUNK,), jnp.int32),
            pltpu.VMEM((EDGE_CHUNK, F_E), jnp.float32),
            pltpu.VMEM((EDGE_CHUNK,), jnp.int32),
            pltpu.VMEM((EDGE_CHUNK // SUB, SUB), jnp.int32),
            pltpu.VMEM((EDGE_CHUNK, CNT_W), jnp.float32),
            pltpu.VMEM_SHARED((G, F_X), jnp.float32),
            pltpu.VMEM_SHARED((G, F_E), jnp.float32),
            pltpu.VMEM_SHARED((G, CNT_W), jnp.float32),
            pltpu.VMEM_SHARED((G, CNT_W), jnp.float32),
        ],
    )
    def agg(x3d_h, batchf_h, row2d_h, ea3d_h, ones_h,
            z256_h, z16_h, z8_h,
            xsum_h, esum_h, ncnt_h, ecnt_h,
            batch_v, xdata, bidx, edata, rowv, gidx, ones_v,
            sh_xsum, sh_esum, sh_ncnt, sh_ecnt):
        cid = lax.axis_index("c")
        sid = lax.axis_index("s")
        wid = sid * NC + cid

        pltpu.sync_copy(batchf_h, batch_v)
        pltpu.sync_copy(ones_h, ones_v)

        @pl.when(sid == 0)
        def _init():
            pltpu.sync_copy(z256_h, sh_xsum)
            pltpu.sync_copy(z16_h, sh_esum)
            pltpu.sync_copy(z8_h, sh_ncnt)
            pltpu.sync_copy(z8_h, sh_ecnt)

        plsc.subcore_barrier()

        for k_i in range(CHUNKS_PER_TILE):
            c = wid + NW * k_i

            @pl.when(c < NUM_CHUNKS)
            def _chunk(c=c):
                pltpu.sync_copy(batchf_h.at[pl.ds(c * NODE_CHUNK, NODE_CHUNK)], bidx)
                pltpu.sync_copy(x3d_h.at[c], xdata)
                pltpu.sync_copy(xdata, sh_xsum.at[bidx], add=True)
                pltpu.sync_copy(ones_v.at[pl.ds(0, NODE_CHUNK)],
                                sh_ncnt.at[bidx], add=True)

                pltpu.sync_copy(row2d_h.at[c], rowv)
                pltpu.sync_copy(ea3d_h.at[c], edata)
                for j in range(EDGE_CHUNK // 16):
                    g16 = plsc.load_gather(batch_v, [rowv[pl.ds(j * 16, 16)]])
                    gidx[j // (SUB // 16), pl.ds((j % (SUB // 16)) * 16, 16)] = g16
                for s in range(EDGE_CHUNK // SUB):
                    pltpu.sync_copy(edata.at[pl.ds(s * SUB, SUB)],
                                    sh_esum.at[gidx.at[s]], add=True)
                    pltpu.sync_copy(ones_v.at[pl.ds(s * SUB, SUB)],
                                    sh_ecnt.at[gidx.at[s]], add=True)

        plsc.subcore_barrier()

        @pl.when(sid == 0)
        def _writeout():
            pltpu.sync_copy(sh_xsum, xsum_h.at[cid])
            pltpu.sync_copy(sh_esum, esum_h.at[cid])
            pltpu.sync_copy(sh_ncnt, ncnt_h.at[cid])
            pltpu.sync_copy(sh_ecnt, ecnt_h.at[cid])

    return agg(x3d, batch_flat, row2d, ea3d, ones, z256, z16, z8)


def _tc_head(xsum2, esum2, ncnt2, ecnt2, u, w1x, w1e, w1u, b1, w2, b2):
    def body(xs_ref, es_ref, nc_ref, ec_ref, u_ref, w1x_ref, w1e_ref,
             w1u_ref, b1_ref, w2_ref, b2_ref, out_ref):
        xsum = xs_ref[0] + xs_ref[1]
        esum = es_ref[0] + es_ref[1]
        ncnt = nc_ref[0] + nc_ref[1]
        ecnt = ec_ref[0] + ec_ref[1]
        xagg = xsum / jnp.maximum(ncnt[:, 0:1], 1.0)
        eagg = esum / jnp.maximum(ecnt[:, 0:1], 1.0)
        h = (jnp.dot(xagg, w1x_ref[...], preferred_element_type=jnp.float32)
             + jnp.dot(eagg, w1e_ref[...], preferred_element_type=jnp.float32)
             + jnp.dot(u_ref[...], w1u_ref[...], preferred_element_type=jnp.float32)
             + b1_ref[...])
        h = jnp.maximum(h, 0.0)
        out_ref[...] = (jnp.dot(h, w2_ref[...], preferred_element_type=jnp.float32)
                        + b2_ref[...])

    return pl.pallas_call(
        body,
        out_shape=jax.ShapeDtypeStruct((G, F_U), jnp.float32),
    )(xsum2, esum2, ncnt2, ecnt2, u, w1x, w1e, w1u, b1, w2, b2)


def kernel(x, edge_index, edge_attr, u, batch, W1, b1, W2, b2):
    x3d = x.reshape(NUM_CHUNKS, NODE_CHUNK, F_X)
    row2d = edge_index[0].reshape(NUM_CHUNKS, EDGE_CHUNK)
    ea3d = edge_attr.reshape(NUM_CHUNKS, EDGE_CHUNK, F_E)
    ones = jnp.ones((EDGE_CHUNK, CNT_W), jnp.float32)
    z256 = jnp.zeros((G, F_X), jnp.float32)
    z16 = jnp.zeros((G, F_E), jnp.float32)
    z8 = jnp.zeros((G, CNT_W), jnp.float32)

    xsum2, esum2, ncnt2, ecnt2 = _sc_aggregate(
        x3d, batch, row2d, ea3d, ones, z256, z16, z8)

    return _tc_head(
        xsum2, esum2, ncnt2, ecnt2, u,
        W1[:F_X], W1[F_X:F_X + F_E], W1[F_X + F_E:],
        b1.reshape(1, H), W2, b2.reshape(1, F_U))


# speedup vs baseline: 11.6449x; 1.0929x over previous
import functools

import jax
import jax.numpy as jnp
from jax import lax
from jax.experimental import pallas as pl
from jax.experimental.pallas import tpu as pltpu
from jax.experimental.pallas import tpu_sc as plsc

G = 16
N_NODES = 10000
N_EDGES = 160000
F_X = 256
F_E = 16
F_U = 256
H = 512

NODE_CHUNK = 80
EDGE_CHUNK = 1280
NUM_CHUNKS = 125
SUB = 128
CNT_W = 8
NBUF = 2

NC = 2
NS = 16
NW = NC * NS
CHUNKS_PER_TILE = (NUM_CHUNKS + NW - 1) // NW


def _sc_aggregate(x3d, batch_flat, ei4, ea3d, ones, z256, z16, z8):
    mesh = plsc.VectorSubcoreMesh(core_axis_name="c", subcore_axis_name="s")

    @functools.partial(
        pl.kernel,
        out_type=(
            jax.ShapeDtypeStruct((NC, G, F_X), jnp.float32),
            jax.ShapeDtypeStruct((NC, G, F_E), jnp.float32),
            jax.ShapeDtypeStruct((NC, G, CNT_W), jnp.float32),
            jax.ShapeDtypeStruct((NC, G, CNT_W), jnp.float32),
        ),
        mesh=mesh,
        compiler_params=pltpu.CompilerParams(
            needs_layout_passes=False, use_tc_tiling_on_sc=False),
        scratch_types=[
            pltpu.VMEM((N_NODES,), jnp.int32),
            pltpu.VMEM((NBUF, NODE_CHUNK, F_X), jnp.float32),
            pltpu.VMEM((NBUF, NODE_CHUNK), jnp.int32),
            pltpu.VMEM((NBUF, EDGE_CHUNK, F_E), jnp.float32),
            pltpu.VMEM((NBUF, EDGE_CHUNK // SUB, SUB), jnp.int32),
            pltpu.VMEM((NBUF, EDGE_CHUNK // SUB, SUB), jnp.int32),
            pltpu.VMEM((EDGE_CHUNK, CNT_W), jnp.float32),
            pltpu.VMEM_SHARED((G + 1, F_X), jnp.float32),
            pltpu.VMEM_SHARED((G + 1, F_E), jnp.float32),
            pltpu.VMEM_SHARED((G + 1, CNT_W), jnp.float32),
            pltpu.VMEM_SHARED((G + 1, CNT_W), jnp.float32),
            pltpu.SemaphoreType.DMA,
            pltpu.SemaphoreType.DMA,
            pltpu.SemaphoreType.DMA,
            pltpu.SemaphoreType.DMA,
            pltpu.SemaphoreType.DMA,
        ],
    )
    def agg(x3d_h, batchf_h, ei4_h, ea3d_h, ones_h, z256_h, z16_h, z8_h,
            xsum_h, esum_h, ncnt_h, ecnt_h,
            batch_v, xdata, bidx, edata, rowv, gidx, ones_v,
            sh_xsum, sh_esum, sh_ncnt, sh_ecnt,
            pre_sem, in_sem0, in_sem1, sc_sem0, sc_sem1):
        in_sems = [in_sem0, in_sem1]
        sc_sems = [sc_sem0, sc_sem1]
        cid = lax.axis_index("c")
        sid = lax.axis_index("s")
        wid = sid * NC + cid

        d_batch = pltpu.async_copy(batchf_h, batch_v, pre_sem)
        d_ones = pltpu.async_copy(ones_h, ones_v, pre_sem)

        @pl.when(sid == 0)
        def _init():
            pltpu.sync_copy(z256_h, sh_xsum)
            pltpu.sync_copy(z16_h, sh_esum)
            pltpu.sync_copy(z8_h, sh_ncnt)
            pltpu.sync_copy(z8_h, sh_ecnt)

        in_descs = {}

        def start_in(k):
            b = k % NBUF
            c = jnp.minimum(wid + NW * k, NUM_CHUNKS - 1)
            sem = in_sems[b]
            in_descs[k] = [
                pltpu.async_copy(
                    batchf_h.at[pl.ds(c * NODE_CHUNK, NODE_CHUNK)],
                    bidx.at[b], sem),
                pltpu.async_copy(x3d_h.at[c], xdata.at[b], sem),
                pltpu.async_copy(ei4_h.at[0, c], rowv.at[b], sem),
                pltpu.async_copy(ea3d_h.at[c], edata.at[b], sem),
            ]

        sc_descs = {}

        def run_chunk(k):
            b = k % NBUF
            for d in in_descs.pop(k):
                d.wait()
            for j in range(EDGE_CHUNK // 16):
                sl = pl.ds((j % (SUB // 16)) * 16, 16)
                g16 = plsc.load_gather(batch_v, [rowv[b, j // (SUB // 16), sl]])
                gidx[b, j // (SUB // 16), sl] = g16
            valid = (wid + NW * k) < NUM_CHUNKS

            @pl.when(jnp.logical_not(valid))
            def _redirect_to_dummy_row():
                dummy = jnp.full((16,), G, jnp.int32)
                for i in range(NODE_CHUNK // 16):
                    bidx[b, pl.ds(i * 16, 16)] = dummy
                for s in range(EDGE_CHUNK // SUB):
                    for i in range(SUB // 16):
                        gidx[b, s, pl.ds(i * 16, 16)] = dummy

            sem = sc_sems[b]
            ds_ = [
                pltpu.async_copy(xdata.at[b], sh_xsum.at[bidx.at[b]],
                                 sem, add=True),
                pltpu.async_copy(ones_v.at[pl.ds(0, NODE_CHUNK)],
                                 sh_ncnt.at[bidx.at[b]], sem, add=True),
            ]
            for s in range(EDGE_CHUNK // SUB):
                ds_.append(pltpu.async_copy(
                    edata.at[b].at[pl.ds(s * SUB, SUB)],
                    sh_esum.at[gidx.at[b, s]], sem, add=True))
                ds_.append(pltpu.async_copy(
                    ones_v.at[pl.ds(s * SUB, SUB)],
                    sh_ecnt.at[gidx.at[b, s]], sem, add=True))
            sc_descs[k] = ds_

        def drain(k):
            for d in sc_descs.pop(k):
                d.wait()

        start_in(0)
        d_batch.wait()
        d_ones.wait()
        plsc.subcore_barrier()

        for k in range(CHUNKS_PER_TILE):
            if k >= 1:
                drain(k - 1)
            if k + 1 < CHUNKS_PER_TILE:
                start_in(k + 1)
            run_chunk(k)
        drain(CHUNKS_PER_TILE - 1)

        plsc.subcore_barrier()

        @pl.when(sid == 0)
        def _writeout():
            pltpu.sync_copy(sh_xsum.at[pl.ds(0, G)], xsum_h.at[cid])
            pltpu.sync_copy(sh_esum.at[pl.ds(0, G)], esum_h.at[cid])
            pltpu.sync_copy(sh_ncnt.at[pl.ds(0, G)], ncnt_h.at[cid])
            pltpu.sync_copy(sh_ecnt.at[pl.ds(0, G)], ecnt_h.at[cid])

    return agg(x3d, batch_flat, ei4, ea3d, ones, z256, z16, z8)


def _tc_head(xsum2, esum2, ncnt2, ecnt2, u, w1, b1, w2, b2):
    def body(xs_ref, es_ref, nc_ref, ec_ref, u_ref, w1_ref,
             b1_ref, w2_ref, b2_ref, out_ref):
        xsum = xs_ref[0] + xs_ref[1]
        esum = es_ref[0] + es_ref[1]
        ncnt = nc_ref[0] + nc_ref[1]
        ecnt = ec_ref[0] + ec_ref[1]
        xagg = xsum / jnp.maximum(ncnt[:, 0:1], 1.0)
        eagg = esum / jnp.maximum(ecnt[:, 0:1], 1.0)
        h = (jnp.dot(xagg, w1_ref[0:F_X, :],
                     preferred_element_type=jnp.float32)
             + jnp.dot(eagg, w1_ref[F_X:F_X + F_E, :],
                       preferred_element_type=jnp.float32)
             + jnp.dot(u_ref[...], w1_ref[F_X + F_E:, :],
                       preferred_element_type=jnp.float32)
             + b1_ref[...])
        h = jnp.maximum(h, 0.0)
        out_ref[...] = (jnp.dot(h, w2_ref[...],
                                preferred_element_type=jnp.float32)
                        + b2_ref[...])

    return pl.pallas_call(
        body,
        out_shape=jax.ShapeDtypeStruct((G, F_U), jnp.float32),
    )(xsum2, esum2, ncnt2, ecnt2, u, w1, b1, w2, b2)


def kernel(x, edge_index, edge_attr, u, batch, W1, b1, W2, b2):
    x3d = x.reshape(NUM_CHUNKS, NODE_CHUNK, F_X)
    ei4 = edge_index.reshape(2, NUM_CHUNKS, EDGE_CHUNK // SUB, SUB)
    ea3d = edge_attr.reshape(NUM_CHUNKS, EDGE_CHUNK, F_E)
    ones = jnp.ones((EDGE_CHUNK, CNT_W), jnp.float32)
    z256 = jnp.zeros((G + 1, F_X), jnp.float32)
    z16 = jnp.zeros((G + 1, F_E), jnp.float32)
    z8 = jnp.zeros((G + 1, CNT_W), jnp.float32)

    xsum2, esum2, ncnt2, ecnt2 = _sc_aggregate(
        x3d, batch, ei4, ea3d, ones, z256, z16, z8)

    return _tc_head(xsum2, esum2, ncnt2, ecnt2, u, W1,
                    b1.reshape(1, H), W2, b2.reshape(1, F_U))


# speedup vs baseline: 12.0246x; 1.0326x over previous
import functools

import jax
import jax.numpy as jnp
from jax import lax
from jax.experimental import pallas as pl
from jax.experimental.pallas import tpu as pltpu
from jax.experimental.pallas import tpu_sc as plsc

G = 16
N_NODES = 10000
N_EDGES = 160000
F_X = 256
F_E = 16
F_U = 256
H = 512

NODE_CHUNK = 80
EDGE_CHUNK = 1280
NUM_CHUNKS = 125
SUB = 128
CNT_W = 8
NBUF = 2

NC = 2
NS = 16
NW = NC * NS
CHUNKS_PER_TILE = (NUM_CHUNKS + NW - 1) // NW


def _sc_aggregate(xt, batch_flat, ei4, ea3d, ones, pat, hpat, z34, z16, z8):
    mesh = plsc.VectorSubcoreMesh(core_axis_name="c", subcore_axis_name="s")

    @functools.partial(
        pl.kernel,
        out_type=(
            jax.ShapeDtypeStruct((NC, 2 * (G + 1), 128), jnp.float32),
            jax.ShapeDtypeStruct((NC, G, F_E), jnp.float32),
            jax.ShapeDtypeStruct((NC, G, CNT_W), jnp.float32),
            jax.ShapeDtypeStruct((NC, G, CNT_W), jnp.float32),
        ),
        mesh=mesh,
        compiler_params=pltpu.CompilerParams(
            needs_layout_passes=False, use_tc_tiling_on_sc=False),
        scratch_types=[
            pltpu.VMEM((N_NODES,), jnp.int32),
            pltpu.VMEM((NBUF, 2 * NODE_CHUNK, 128), jnp.float32),
            pltpu.VMEM((NBUF, NODE_CHUNK), jnp.int32),
            pltpu.VMEM((NBUF, 2, NODE_CHUNK), jnp.int32),
            pltpu.VMEM((2, NODE_CHUNK), jnp.int32),
            pltpu.VMEM((2, NODE_CHUNK), jnp.int32),
            pltpu.VMEM((NBUF, EDGE_CHUNK, F_E), jnp.float32),
            pltpu.VMEM((NBUF, EDGE_CHUNK // SUB, SUB), jnp.int32),
            pltpu.VMEM((NBUF, EDGE_CHUNK // SUB, SUB), jnp.int32),
            pltpu.VMEM((EDGE_CHUNK, CNT_W), jnp.float32),
            pltpu.VMEM_SHARED((2 * (G + 1), 128), jnp.float32),
            pltpu.VMEM_SHARED((G + 1, F_E), jnp.float32),
            pltpu.VMEM_SHARED((G + 1, CNT_W), jnp.float32),
            pltpu.VMEM_SHARED((G + 1, CNT_W), jnp.float32),
            pltpu.SemaphoreType.DMA,
            pltpu.SemaphoreType.DMA,
            pltpu.SemaphoreType.DMA,
            pltpu.SemaphoreType.DMA,
            pltpu.SemaphoreType.DMA,
        ],
    )
    def agg(xt_h, batchf_h, ei4_h, ea3d_h, ones_h, pat_h, hpat_h,
            z34_h, z16_h, z8_h,
            xsum_h, esum_h, ncnt_h, ecnt_h,
            batch_v, xtile, bidx, sidx, pat_v, hpat_v,
            edata, rowv, gidx, ones_v,
            sh_xsum, sh_esum, sh_ncnt, sh_ecnt,
            pre_sem, in_sem0, in_sem1, sc_sem0, sc_sem1):
        in_sems = [in_sem0, in_sem1]
        sc_sems = [sc_sem0, sc_sem1]
        cid = lax.axis_index("c")
        sid = lax.axis_index("s")
        wid = sid * NC + cid

        d_batch = pltpu.async_copy(batchf_h, batch_v, pre_sem)
        d_ones = pltpu.async_copy(ones_h, ones_v, pre_sem)
        d_pat = pltpu.async_copy(pat_h, pat_v, pre_sem)
        d_hpat = pltpu.async_copy(hpat_h, hpat_v, pre_sem)

        @pl.when(sid == 0)
        def _init():
            pltpu.sync_copy(z34_h, sh_xsum)
            pltpu.sync_copy(z16_h, sh_esum)
            pltpu.sync_copy(z8_h, sh_ncnt)
            pltpu.sync_copy(z8_h, sh_ecnt)

        in_descs = {}

        def start_in(k):
            b = k % NBUF
            c = jnp.minimum(wid + NW * k, NUM_CHUNKS - 1)
            sem = in_sems[b]
            in_descs[k] = [
                pltpu.async_copy(
                    batchf_h.at[pl.ds(c * NODE_CHUNK, NODE_CHUNK)],
                    bidx.at[b], sem),
                pltpu.async_copy(xt_h.at[c], xtile.at[b], sem),
                pltpu.async_copy(ei4_h.at[0, c], rowv.at[b], sem),
                pltpu.async_copy(ea3d_h.at[c], edata.at[b], sem),
            ]

        sc_descs = {}

        def run_chunk(k):
            b = k % NBUF
            for d in in_descs.pop(k):
                d.wait()
            for j in range(EDGE_CHUNK // 16):
                sl = pl.ds((j % (SUB // 16)) * 16, 16)
                g16 = plsc.load_gather(batch_v, [rowv[b, j // (SUB // 16), sl]])
                gidx[b, j // (SUB // 16), sl] = g16
            for i in range(2 * NODE_CHUNK // 16):
                r, csl = i // 5, pl.ds((i % 5) * 16, 16)
                ng16 = plsc.load_gather(bidx.at[b], [pat_v[r, csl]])
                sidx[b, r, csl] = 2 * ng16 + hpat_v[r, csl]
            valid = (wid + NW * k) < NUM_CHUNKS

            @pl.when(jnp.logical_not(valid))
            def _redirect_to_dummy_row():
                dummy = jnp.full((16,), G, jnp.int32)
                dummy2 = jnp.full((16,), 2 * G, jnp.int32)
                for i in range(NODE_CHUNK // 16):
                    bidx[b, pl.ds(i * 16, 16)] = dummy
                for i in range(2 * NODE_CHUNK // 16):
                    sidx[b, i // 5, pl.ds((i % 5) * 16, 16)] = dummy2
                for s in range(EDGE_CHUNK // SUB):
                    for i in range(SUB // 16):
                        gidx[b, s, pl.ds(i * 16, 16)] = dummy

            sem = sc_sems[b]
            ds_ = [
                pltpu.async_copy(xtile.at[b].at[pl.ds(0, NODE_CHUNK)],
                                 sh_xsum.at[sidx.at[b, 0]], sem, add=True),
                pltpu.async_copy(xtile.at[b].at[pl.ds(NODE_CHUNK, NODE_CHUNK)],
                                 sh_xsum.at[sidx.at[b, 1]], sem, add=True),
                pltpu.async_copy(ones_v.at[pl.ds(0, NODE_CHUNK)],
                                 sh_ncnt.at[bidx.at[b]], sem, add=True),
            ]
            for s in range(EDGE_CHUNK // SUB):
                ds_.append(pltpu.async_copy(
                    edata.at[b].at[pl.ds(s * SUB, SUB)],
                    sh_esum.at[gidx.at[b, s]], sem, add=True))
                ds_.append(pltpu.async_copy(
                    ones_v.at[pl.ds(s * SUB, SUB)],
                    sh_ecnt.at[gidx.at[b, s]], sem, add=True))
            sc_descs[k] = ds_

        def drain(k):
            for d in sc_descs.pop(k):
                d.wait()

        start_in(0)
        d_batch.wait()
        d_ones.wait()
        d_pat.wait()
        d_hpat.wait()
        plsc.subcore_barrier()

        for k in range(CHUNKS_PER_TILE):
            if k >= 1:
                drain(k - 1)
            if k + 1 < CHUNKS_PER_TILE:
                start_in(k + 1)
            run_chunk(k)
        drain(CHUNKS_PER_TILE - 1)

        plsc.subcore_barrier()

        @pl.when(sid == 0)
        def _writeout():
            pltpu.sync_copy(sh_xsum.at[pl.ds(0, 2 * G)],
                            xsum_h.at[cid].at[pl.ds(0, 2 * G)])
            pltpu.sync_copy(sh_esum.at[pl.ds(0, G)], esum_h.at[cid])
            pltpu.sync_copy(sh_ncnt.at[pl.ds(0, G)], ncnt_h.at[cid])
            pltpu.sync_copy(sh_ecnt.at[pl.ds(0, G)], ecnt_h.at[cid])

    return agg(xt, batch_flat, ei4, ea3d, ones, pat, hpat, z34, z16, z8)


def _tc_head(xsum2, esum2, ncnt2, ecnt2, u, w1, b1, w2, b2):
    def body(xs_ref, es_ref, nc_ref, ec_ref, u_ref, w1_ref,
             b1_ref, w2_ref, b2_ref, out_ref):
        xsum = xs_ref[0] + xs_ref[1]
        esum = es_ref[0] + es_ref[1]
        ncnt = nc_ref[0] + nc_ref[1]
        ecnt = ec_ref[0] + ec_ref[1]
        xagg = xsum / jnp.maximum(ncnt[:, 0:1], 1.0)
        eagg = esum / jnp.maximum(ecnt[:, 0:1], 1.0)
        h = (jnp.dot(xagg, w1_ref[0:F_X, :],
                     preferred_element_type=jnp.float32)
             + jnp.dot(eagg, w1_ref[F_X:F_X + F_E, :],
                       preferred_element_type=jnp.float32)
             + jnp.dot(u_ref[...], w1_ref[F_X + F_E:, :],
                       preferred_element_type=jnp.float32)
             + b1_ref[...])
        h = jnp.maximum(h, 0.0)
        out_ref[...] = (jnp.dot(h, w2_ref[...],
                                preferred_element_type=jnp.float32)
                        + b2_ref[...])

    return pl.pallas_call(
        body,
        out_shape=jax.ShapeDtypeStruct((G, F_U), jnp.float32),
    )(xsum2, esum2, ncnt2, ecnt2, u, w1, b1, w2, b2)


def kernel(x, edge_index, edge_attr, u, batch, W1, b1, W2, b2):
    xt = (x.reshape(N_NODES // 8, 8, 2, 128)
          .transpose(0, 2, 1, 3)
          .reshape(NUM_CHUNKS, 2 * NODE_CHUNK, 128))
    ei4 = edge_index.reshape(2, NUM_CHUNKS, EDGE_CHUNK // SUB, SUB)
    ea3d = edge_attr.reshape(NUM_CHUNKS, EDGE_CHUNK, F_E)
    ones = jnp.ones((EDGE_CHUNK, CNT_W), jnp.float32)
    k_ar = jnp.arange(2 * NODE_CHUNK, dtype=jnp.int32)
    pat = (8 * (k_ar // 16) + k_ar % 8).reshape(2, NODE_CHUNK)
    hpat = ((k_ar // 8) % 2).reshape(2, NODE_CHUNK)
    z34 = jnp.zeros((2 * (G + 1), 128), jnp.float32)
    z16 = jnp.zeros((G + 1, F_E), jnp.float32)
    z8 = jnp.zeros((G + 1, CNT_W), jnp.float32)

    xsum2h, esum2, ncnt2, ecnt2 = _sc_aggregate(
        xt, batch, ei4, ea3d, ones, pat, hpat, z34, z16, z8)

    xsum2 = xsum2h.reshape(NC, G + 1, 2, 128)[:, :G].reshape(NC, G, F_X)

    return _tc_head(xsum2, esum2, ncnt2, ecnt2, u, W1,
                    b1.reshape(1, H), W2, b2.reshape(1, F_U))


# speedup vs baseline: 12.8494x; 1.0686x over previous
import functools

import jax
import jax.numpy as jnp
from jax import lax
from jax.experimental import pallas as pl
from jax.experimental.pallas import tpu as pltpu
from jax.experimental.pallas import tpu_sc as plsc

G = 16
N_NODES = 10000
N_EDGES = 160000
F_X = 256
F_E = 16
F_U = 256
H = 512

NODE_CHUNK = 40
EDGE_CHUNK = 640
NUM_CHUNKS = 250
EB = EDGE_CHUNK // 128
XROWS = 2 * NODE_CHUNK
SUB = 128
CNT_W = 8
NBUF = 2

NC = 2
NS = 16
NW = NC * NS
CHUNKS_PER_TILE = (NUM_CHUNKS + NW - 1) // NW


def _sc_aggregate(xt, batch_flat, ei4, eav, ones, pat, hpat, z34, z16, z8):
    mesh = plsc.VectorSubcoreMesh(core_axis_name="c", subcore_axis_name="s")

    @functools.partial(
        pl.kernel,
        out_type=(
            jax.ShapeDtypeStruct((NC, 2 * (G + 1), 128), jnp.float32),
            jax.ShapeDtypeStruct((NC, G, F_E), jnp.float32),
            jax.ShapeDtypeStruct((NC, G, CNT_W), jnp.float32),
            jax.ShapeDtypeStruct((NC, G, CNT_W), jnp.float32),
        ),
        mesh=mesh,
        compiler_params=pltpu.CompilerParams(
            needs_layout_passes=False, use_tc_tiling_on_sc=False),
        scratch_types=[
            pltpu.VMEM((N_NODES,), jnp.int32),
            pltpu.VMEM((NBUF, XROWS, 128), jnp.float32),
            pltpu.VMEM((NBUF, NODE_CHUNK), jnp.int32),
            pltpu.VMEM((NBUF, XROWS), jnp.int32),
            pltpu.VMEM((XROWS,), jnp.int32),
            pltpu.VMEM((XROWS,), jnp.int32),
            pltpu.VMEM((NBUF, 2, EB, 8, 128), jnp.float32),
            pltpu.VMEM((NBUF, EDGE_CHUNK, F_E), jnp.float32),
            pltpu.VMEM((NBUF, EB, 128), jnp.int32),
            pltpu.VMEM((NBUF, EB, 128), jnp.int32),
            pltpu.VMEM((SUB, CNT_W), jnp.float32),
            pltpu.VMEM_SHARED((2 * (G + 1), 128), jnp.float32),
            pltpu.VMEM_SHARED((G + 1, F_E), jnp.float32),
            pltpu.VMEM_SHARED((G + 1, CNT_W), jnp.float32),
            pltpu.VMEM_SHARED((G + 1, CNT_W), jnp.float32),
            pltpu.SemaphoreType.DMA,
            pltpu.SemaphoreType.DMA,
            pltpu.SemaphoreType.DMA,
            pltpu.SemaphoreType.DMA,
            pltpu.SemaphoreType.DMA,
        ],
    )
    def agg(xt_h, batchf_h, ei4_h, eav_h, ones_h, pat_h, hpat_h,
            z34_h, z16_h, z8_h,
            xsum_h, esum_h, ncnt_h, ecnt_h,
            batch_v, xtile, bidx, sidx, pat_v, hpat_v,
            enat, edata, rowv, gidx, ones_v,
            sh_xsum, sh_esum, sh_ncnt, sh_ecnt,
            pre_sem, in_sem0, in_sem1, sc_sem0, sc_sem1):
        in_sems = [in_sem0, in_sem1]
        sc_sems = [sc_sem0, sc_sem1]
        cid = lax.axis_index("c")
        sid = lax.axis_index("s")
        wid = sid * NC + cid

        iota16 = lax.iota(jnp.int32, 16)
        i_fb = iota16 // 8
        i_s = iota16 % 8

        d_batch = pltpu.async_copy(batchf_h, batch_v, pre_sem)
        d_ones = pltpu.async_copy(ones_h, ones_v, pre_sem)
        d_pat = pltpu.async_copy(pat_h, pat_v, pre_sem)
        d_hpat = pltpu.async_copy(hpat_h, hpat_v, pre_sem)

        @pl.when(sid == 0)
        def _init():
            pltpu.sync_copy(z34_h, sh_xsum)
            pltpu.sync_copy(z16_h, sh_esum)
            pltpu.sync_copy(z8_h, sh_ncnt)
            pltpu.sync_copy(z8_h, sh_ecnt)

        in_descs = {}

        def start_in(k):
            b = k % NBUF
            c = jnp.minimum(wid + NW * k, NUM_CHUNKS - 1)
            sem = in_sems[b]
            in_descs[k] = [
                pltpu.async_copy(
                    batchf_h.at[pl.ds(c * NODE_CHUNK, NODE_CHUNK)],
                    bidx.at[b], sem),
                pltpu.async_copy(xt_h.at[c], xtile.at[b], sem),
                pltpu.async_copy(ei4_h.at[0, c], rowv.at[b], sem),
                pltpu.async_copy(eav_h.at[0].at[pl.ds(c * EB, EB)],
                                 enat.at[b, 0], sem),
                pltpu.async_copy(eav_h.at[1].at[pl.ds(c * EB, EB)],
                                 enat.at[b, 1], sem),
            ]

        sc_descs = {}

        def run_chunk(k):
            b = k % NBUF
            for d in in_descs.pop(k):
                d.wait()

            for j in range(EDGE_CHUNK // 16):
                sl = pl.ds((j % (SUB // 16)) * 16, 16)
                g16 = plsc.load_gather(batch_v, [rowv[b, j // (SUB // 16), sl]])
                gidx[b, j // (SUB // 16), sl] = g16

            def dt_body(it, _):
                ce0 = it * 16
                for j in range(16):
                    ce = ce0 + j
                    v = plsc.load_gather(
                        enat.at[b],
                        [i_fb, jnp.zeros((16,), jnp.int32) + ce // 128,
                         i_s, jnp.zeros((16,), jnp.int32) + ce % 128])
                    edata[b, ce, :] = v
                return 0

            lax.fori_loop(0, EDGE_CHUNK // 16, dt_body, 0)

            for i in range(XROWS // 16):
                sl = pl.ds(i * 16, 16)
                ng16 = plsc.load_gather(bidx.at[b], [pat_v[sl]])
                sidx[b, sl] = 2 * ng16 + hpat_v[sl]

            valid = (wid + NW * k) < NUM_CHUNKS

            @pl.when(jnp.logical_not(valid))
            def _redirect_to_dummy_row():
                dummy = jnp.full((16,), G, jnp.int32)
                dummy2 = jnp.full((16,), 2 * G, jnp.int32)
                for i in range((NODE_CHUNK + 15) // 16):
                    bidx[b, pl.ds(min(i * 16, NODE_CHUNK - 16), 16)] = dummy
                for i in range(XROWS // 16):
                    sidx[b, pl.ds(i * 16, 16)] = dummy2
                for s in range(EB):
                    for i in range(SUB // 16):
                        gidx[b, s, pl.ds(i * 16, 16)] = dummy

            sem = sc_sems[b]
            ds_ = [
                pltpu.async_copy(xtile.at[b], sh_xsum.at[sidx.at[b]],
                                 sem, add=True),
                pltpu.async_copy(ones_v.at[pl.ds(0, NODE_CHUNK)],
                                 sh_ncnt.at[bidx.at[b]], sem, add=True),
            ]
            for s in range(EB):
                ds_.append(pltpu.async_copy(
                    edata.at[b].at[pl.ds(s * SUB, SUB)],
                    sh_esum.at[gidx.at[b, s]], sem, add=True))
                ds_.append(pltpu.async_copy(
                    ones_v, sh_ecnt.at[gidx.at[b, s]], sem, add=True))
            sc_descs[k] = ds_

        def drain(k):
            for d in sc_descs.pop(k):
                d.wait()

        start_in(0)
        d_batch.wait()
        d_ones.wait()
        d_pat.wait()
        d_hpat.wait()
        plsc.subcore_barrier()

        for k in range(CHUNKS_PER_TILE):
            if k >= 1:
                drain(k - 1)
            if k + 1 < CHUNKS_PER_TILE:
                start_in(k + 1)
            run_chunk(k)
        drain(CHUNKS_PER_TILE - 1)

        plsc.subcore_barrier()

        @pl.when(sid == 0)
        def _writeout():
            pltpu.sync_copy(sh_xsum.at[pl.ds(0, 2 * G)],
                            xsum_h.at[cid].at[pl.ds(0, 2 * G)])
            pltpu.sync_copy(sh_esum.at[pl.ds(0, G)], esum_h.at[cid])
            pltpu.sync_copy(sh_ncnt.at[pl.ds(0, G)], ncnt_h.at[cid])
            pltpu.sync_copy(sh_ecnt.at[pl.ds(0, G)], ecnt_h.at[cid])

    return agg(xt, batch_flat, ei4, eav, ones, pat, hpat, z34, z16, z8)


def _tc_head(xsum2, esum2, ncnt2, ecnt2, u, w1, b1, w2, b2):
    def body(xs_ref, es_ref, nc_ref, ec_ref, u_ref, w1_ref,
             b1_ref, w2_ref, b2_ref, out_ref):
        xsum = xs_ref[0] + xs_ref[1]
        esum = es_ref[0] + es_ref[1]
        ncnt = nc_ref[0] + nc_ref[1]
        ecnt = ec_ref[0] + ec_ref[1]
        xagg = xsum / jnp.maximum(ncnt[:, 0:1], 1.0)
        eagg = esum / jnp.maximum(ecnt[:, 0:1], 1.0)
        h = (jnp.dot(xagg, w1_ref[0:F_X, :],
                     preferred_element_type=jnp.float32)
             + jnp.dot(eagg, w1_ref[F_X:F_X + F_E, :],
                       preferred_element_type=jnp.float32)
             + jnp.dot(u_ref[...], w1_ref[F_X + F_E:, :],
                       preferred_element_type=jnp.float32)
             + b1_ref[...])
        h = jnp.maximum(h, 0.0)
        out_ref[...] = (jnp.dot(h, w2_ref[...],
                                preferred_element_type=jnp.float32)
                        + b2_ref[...])

    return pl.pallas_call(
        body,
        out_shape=jax.ShapeDtypeStruct((G, F_U), jnp.float32),
    )(xsum2, esum2, ncnt2, ecnt2, u, w1, b1, w2, b2)


def kernel(x, edge_index, edge_attr, u, batch, W1, b1, W2, b2):
    xt = (x.reshape(N_NODES // 8, 8, 2, 128)
          .transpose(0, 2, 1, 3)
          .reshape(NUM_CHUNKS, XROWS, 128))
    eav = (edge_attr.T.reshape(2, 8, N_EDGES // 128, 128)
           .transpose(0, 2, 1, 3))
    ei4 = edge_index.reshape(2, NUM_CHUNKS, EB, 128)
    ones = jnp.ones((SUB, CNT_W), jnp.float32)
    k_ar = jnp.arange(XROWS, dtype=jnp.int32)
    pat = 8 * (k_ar // 16) + k_ar % 8
    hpat = (k_ar // 8) % 2
    z34 = jnp.zeros((2 * (G + 1), 128), jnp.float32)
    z16 = jnp.zeros((G + 1, F_E), jnp.float32)
    z8 = jnp.zeros((G + 1, CNT_W), jnp.float32)

    xsum2h, esum2, ncnt2, ecnt2 = _sc_aggregate(
        xt, batch, ei4, eav, ones, pat, hpat, z34, z16, z8)

    xsum2 = xsum2h.reshape(NC, G + 1, 2, 128)[:, :G].reshape(NC, G, F_X)

    return _tc_head(xsum2, esum2, ncnt2, ecnt2, u, W1,
                    b1.reshape(1, H), W2, b2.reshape(1, F_U))


# speedup vs baseline: 18.4342x; 1.4346x over previous
import functools

import jax
import jax.numpy as jnp
from jax import lax
from jax.experimental import pallas as pl
from jax.experimental.pallas import tpu as pltpu
from jax.experimental.pallas import tpu_sc as plsc

G = 16
N_NODES = 10000
N_EDGES = 160000
F_X = 256
F_E = 16
F_U = 256
H = 512

NODE_CHUNK = 40
EDGE_CHUNK = 640
NUM_CHUNKS = 250
EB = EDGE_CHUNK // 128
XROWS = 2 * NODE_CHUNK
SUB = 128
CNT_W = 8
NBUF = 2

NC = 2
NS = 16
NW = NC * NS
CHUNKS_PER_TILE = (NUM_CHUNKS + NW - 1) // NW


def _sc_aggregate(xt, batch_flat, ei4, eav, ones, pat, hpat, z34, z16, z8):
    mesh = plsc.VectorSubcoreMesh(core_axis_name="c", subcore_axis_name="s")

    @functools.partial(
        pl.kernel,
        out_type=(
            jax.ShapeDtypeStruct((NC, 2 * (G + 1), 128), jnp.float32),
            jax.ShapeDtypeStruct((NC, G, F_E), jnp.float32),
            jax.ShapeDtypeStruct((NC, G, CNT_W), jnp.float32),
            jax.ShapeDtypeStruct((NC, G, CNT_W), jnp.float32),
        ),
        mesh=mesh,
        compiler_params=pltpu.CompilerParams(
            needs_layout_passes=False, use_tc_tiling_on_sc=False),
        scratch_types=[
            pltpu.VMEM((N_NODES,), jnp.int32),
            pltpu.VMEM((NBUF, XROWS, 128), jnp.float32),
            pltpu.VMEM((NBUF, NODE_CHUNK), jnp.int32),
            pltpu.VMEM((NBUF, XROWS), jnp.int32),
            pltpu.VMEM((XROWS,), jnp.int32),
            pltpu.VMEM((XROWS,), jnp.int32),
            pltpu.VMEM((NBUF, 16, EDGE_CHUNK), jnp.float32),
            pltpu.VMEM((NBUF, EDGE_CHUNK, F_E), jnp.float32),
            pltpu.VMEM((NBUF, EB, 128), jnp.int32),
            pltpu.VMEM((NBUF, EB, 128), jnp.int32),
            pltpu.VMEM((SUB, CNT_W), jnp.float32),
            pltpu.VMEM_SHARED((2 * (G + 1), 128), jnp.float32),
            pltpu.VMEM_SHARED((G + 1, F_E), jnp.float32),
            pltpu.VMEM_SHARED((G + 1, CNT_W), jnp.float32),
            pltpu.VMEM_SHARED((G + 1, CNT_W), jnp.float32),
            pltpu.SemaphoreType.DMA,
            pltpu.SemaphoreType.DMA,
            pltpu.SemaphoreType.DMA,
            pltpu.SemaphoreType.DMA,
            pltpu.SemaphoreType.DMA,
        ],
    )
    def agg(xt_h, batchf_h, ei4_h, eav_h, ones_h, pat_h, hpat_h,
            z34_h, z16_h, z8_h,
            xsum_h, esum_h, ncnt_h, ecnt_h,
            batch_v, xtile, bidx, sidx, pat_v, hpat_v,
            enat2, edata, rowv, gidx, ones_v,
            sh_xsum, sh_esum, sh_ncnt, sh_ecnt,
            pre_sem, in_sem0, in_sem1, sc_sem0, sc_sem1):
        in_sems = [in_sem0, in_sem1]
        sc_sems = [sc_sem0, sc_sem1]
        cid = lax.axis_index("c")
        sid = lax.axis_index("s")
        wid = sid * NC + cid

        iota16 = lax.iota(jnp.int32, 16)

        d_batch = pltpu.async_copy(batchf_h, batch_v, pre_sem)
        d_ones = pltpu.async_copy(ones_h, ones_v, pre_sem)
        d_pat = pltpu.async_copy(pat_h, pat_v, pre_sem)
        d_hpat = pltpu.async_copy(hpat_h, hpat_v, pre_sem)

        @pl.when(sid == 0)
        def _init():
            pltpu.sync_copy(z34_h, sh_xsum)
            pltpu.sync_copy(z16_h, sh_esum)
            pltpu.sync_copy(z8_h, sh_ncnt)
            pltpu.sync_copy(z8_h, sh_ecnt)

        in_descs = {}

        def start_in(k):
            b = k % NBUF
            c = jnp.minimum(wid + NW * k, NUM_CHUNKS - 1)
            sem = in_sems[b]
            in_descs[k] = [
                pltpu.async_copy(
                    batchf_h.at[pl.ds(c * NODE_CHUNK, NODE_CHUNK)],
                    bidx.at[b], sem),
                pltpu.async_copy(xt_h.at[c], xtile.at[b], sem),
                pltpu.async_copy(ei4_h.at[0, c], rowv.at[b], sem),
            ] + [
                pltpu.async_copy(
                    eav_h.at[fb, c * EB + eb],
                    enat2.at[b, pl.ds(fb * 8, 8), pl.ds(eb * 128, 128)],
                    sem)
                for fb in range(2) for eb in range(EB)
            ]

        sc_descs = {}

        def run_chunk(k):
            b = k % NBUF
            for d in in_descs.pop(k):
                d.wait()

            for j in range(EDGE_CHUNK // 16):
                sl = pl.ds((j % (SUB // 16)) * 16, 16)
                g16 = plsc.load_gather(batch_v, [rowv[b, j // (SUB // 16), sl]])
                gidx[b, j // (SUB // 16), sl] = g16

            def dt_body(it, _):
                ce0 = it * 16
                ipos = iota16 + ce0
                for r in range(16):
                    fr = (iota16 + r) & 15
                    v = plsc.load_gather(enat2.at[b], [fr, ipos])
                    plsc.store_scatter(edata.at[b], [ipos, fr], v)
                return 0

            lax.fori_loop(0, EDGE_CHUNK // 16, dt_body, 0)

            for i in range(XROWS // 16):
                sl = pl.ds(i * 16, 16)
                ng16 = plsc.load_gather(bidx.at[b], [pat_v[sl]])
                sidx[b, sl] = 2 * ng16 + hpat_v[sl]

            valid = (wid + NW * k) < NUM_CHUNKS

            @pl.when(jnp.logical_not(valid))
            def _redirect_to_dummy_row():
                dummy = jnp.full((16,), G, jnp.int32)
                dummy2 = jnp.full((16,), 2 * G, jnp.int32)
                for i in range((NODE_CHUNK + 15) // 16):
                    bidx[b, pl.ds(min(i * 16, NODE_CHUNK - 16), 16)] = dummy
                for i in range(XROWS // 16):
                    sidx[b, pl.ds(i * 16, 16)] = dummy2
                for s in range(EB):
                    for i in range(SUB // 16):
                        gidx[b, s, pl.ds(i * 16, 16)] = dummy

            sem = sc_sems[b]
            ds_ = [
                pltpu.async_copy(xtile.at[b], sh_xsum.at[sidx.at[b]],
                                 sem, add=True),
                pltpu.async_copy(ones_v.at[pl.ds(0, NODE_CHUNK)],
                                 sh_ncnt.at[bidx.at[b]], sem, add=True),
            ]
            for s in range(EB):
                ds_.append(pltpu.async_copy(
                    edata.at[b].at[pl.ds(s * SUB, SUB)],
                    sh_esum.at[gidx.at[b, s]], sem, add=True))
                ds_.append(pltpu.async_copy(
                    ones_v, sh_ecnt.at[gidx.at[b, s]], sem, add=True))
            sc_descs[k] = ds_

        def drain(k):
            for d in sc_descs.pop(k):
                d.wait()

        start_in(0)
        d_batch.wait()
        d_ones.wait()
        d_pat.wait()
        d_hpat.wait()
        plsc.subcore_barrier()

        for k in range(CHUNKS_PER_TILE):
            if k >= 1:
                drain(k - 1)
            if k + 1 < CHUNKS_PER_TILE:
                start_in(k + 1)
            run_chunk(k)
        drain(CHUNKS_PER_TILE - 1)

        plsc.subcore_barrier()

        @pl.when(sid == 0)
        def _writeout():
            pltpu.sync_copy(sh_xsum.at[pl.ds(0, 2 * G)],
                            xsum_h.at[cid].at[pl.ds(0, 2 * G)])
            pltpu.sync_copy(sh_esum.at[pl.ds(0, G)], esum_h.at[cid])
            pltpu.sync_copy(sh_ncnt.at[pl.ds(0, G)], ncnt_h.at[cid])
            pltpu.sync_copy(sh_ecnt.at[pl.ds(0, G)], ecnt_h.at[cid])

    return agg(xt, batch_flat, ei4, eav, ones, pat, hpat, z34, z16, z8)


def _tc_head(xsum2, esum2, ncnt2, ecnt2, u, w1, b1, w2, b2):
    def body(xs_ref, es_ref, nc_ref, ec_ref, u_ref, w1_ref,
             b1_ref, w2_ref, b2_ref, out_ref):
        xsum = xs_ref[0] + xs_ref[1]
        esum = es_ref[0] + es_ref[1]
        ncnt = nc_ref[0] + nc_ref[1]
        ecnt = ec_ref[0] + ec_ref[1]
        xagg = xsum / jnp.maximum(ncnt[:, 0:1], 1.0)
        eagg = esum / jnp.maximum(ecnt[:, 0:1], 1.0)
        h = (jnp.dot(xagg, w1_ref[0:F_X, :],
                     preferred_element_type=jnp.float32)
             + jnp.dot(eagg, w1_ref[F_X:F_X + F_E, :],
                       preferred_element_type=jnp.float32)
             + jnp.dot(u_ref[...], w1_ref[F_X + F_E:, :],
                       preferred_element_type=jnp.float32)
             + b1_ref[...])
        h = jnp.maximum(h, 0.0)
        out_ref[...] = (jnp.dot(h, w2_ref[...],
                                preferred_element_type=jnp.float32)
                        + b2_ref[...])

    return pl.pallas_call(
        body,
        out_shape=jax.ShapeDtypeStruct((G, F_U), jnp.float32),
    )(xsum2, esum2, ncnt2, ecnt2, u, w1, b1, w2, b2)


def kernel(x, edge_index, edge_attr, u, batch, W1, b1, W2, b2):
    xt = (x.reshape(N_NODES // 8, 8, 2, 128)
          .transpose(0, 2, 1, 3)
          .reshape(NUM_CHUNKS, XROWS, 128))
    eav = (edge_attr.T.reshape(2, 8, N_EDGES // 128, 128)
           .transpose(0, 2, 1, 3))
    ei4 = edge_index.reshape(2, NUM_CHUNKS, EB, 128)
    ones = jnp.ones((SUB, CNT_W), jnp.float32)
    k_ar = jnp.arange(XROWS, dtype=jnp.int32)
    pat = 8 * (k_ar // 16) + k_ar % 8
    hpat = (k_ar // 8) % 2
    z34 = jnp.zeros((2 * (G + 1), 128), jnp.float32)
    z16 = jnp.zeros((G + 1, F_E), jnp.float32)
    z8 = jnp.zeros((G + 1, CNT_W), jnp.float32)

    xsum2h, esum2, ncnt2, ecnt2 = _sc_aggregate(
        xt, batch, ei4, eav, ones, pat, hpat, z34, z16, z8)

    xsum2 = xsum2h.reshape(NC, G + 1, 2, 128)[:, :G].reshape(NC, G, F_X)

    return _tc_head(xsum2, esum2, ncnt2, ecnt2, u, W1,
                    b1.reshape(1, H), W2, b2.reshape(1, F_U))


# speedup vs baseline: 18.4750x; 1.0022x over previous
import functools

import jax
import jax.numpy as jnp
from jax import lax
from jax.experimental import pallas as pl
from jax.experimental.pallas import tpu as pltpu
from jax.experimental.pallas import tpu_sc as plsc

G = 16
N_NODES = 10000
N_EDGES = 160000
F_X = 256
F_E = 16
F_U = 256
H = 512

NODE_CHUNK = 40
EDGE_CHUNK = 640
NUM_CHUNKS = 250
EB = EDGE_CHUNK // 128
XROWS = 2 * NODE_CHUNK
SUB = 128
CNT_W = 8
NBUF = 2

NC = 2
NS = 16
NW = NC * NS
CHUNKS_PER_TILE = (NUM_CHUNKS + NW - 1) // NW


def _sc_aggregate(xt, batch_flat, ei4, eav, ones, pat, hpat, z34, z16, z8):
    mesh = plsc.VectorSubcoreMesh(core_axis_name="c", subcore_axis_name="s")

    @functools.partial(
        pl.kernel,
        out_type=(
            jax.ShapeDtypeStruct((NC, 2 * (G + 1), 128), jnp.float32),
            jax.ShapeDtypeStruct((NC, G, F_E), jnp.float32),
            jax.ShapeDtypeStruct((NC, G, CNT_W), jnp.float32),
            jax.ShapeDtypeStruct((NC, G, CNT_W), jnp.float32),
        ),
        mesh=mesh,
        compiler_params=pltpu.CompilerParams(
            needs_layout_passes=False, use_tc_tiling_on_sc=False),
        scratch_types=[
            pltpu.VMEM((N_NODES,), jnp.int32),
            pltpu.VMEM((NBUF, XROWS, 128), jnp.float32),
            pltpu.VMEM((NBUF, NODE_CHUNK), jnp.int32),
            pltpu.VMEM((NBUF, XROWS), jnp.int32),
            pltpu.VMEM((XROWS,), jnp.int32),
            pltpu.VMEM((XROWS,), jnp.int32),
            pltpu.VMEM((NBUF, 16, EDGE_CHUNK), jnp.float32),
            pltpu.VMEM((NBUF, EDGE_CHUNK, F_E), jnp.float32),
            pltpu.VMEM((NBUF, EB, 128), jnp.int32),
            pltpu.VMEM((NBUF, EB, 128), jnp.int32),
            pltpu.VMEM((SUB, CNT_W), jnp.float32),
            pltpu.VMEM_SHARED((2 * (G + 1), 128), jnp.float32),
            pltpu.VMEM_SHARED((G + 1, F_E), jnp.float32),
            pltpu.VMEM_SHARED((G + 1, CNT_W), jnp.float32),
            pltpu.VMEM_SHARED((G + 1, CNT_W), jnp.float32),
            pltpu.SemaphoreType.DMA,
            pltpu.SemaphoreType.DMA,
            pltpu.SemaphoreType.DMA,
            pltpu.SemaphoreType.DMA,
            pltpu.SemaphoreType.DMA,
        ],
    )
    def agg(xt_h, batchf_h, ei4_h, eav_h, ones_h, pat_h, hpat_h,
            z34_h, z16_h, z8_h,
            xsum_h, esum_h, ncnt_h, ecnt_h,
            batch_v, xtile, bidx, sidx, pat_v, hpat_v,
            enat2, edata, rowv, gidx, ones_v,
            sh_xsum, sh_esum, sh_ncnt, sh_ecnt,
            pre_sem, in_sem0, in_sem1, sc_sem0, sc_sem1):
        in_sems = [in_sem0, in_sem1]
        sc_sems = [sc_sem0, sc_sem1]
        cid = lax.axis_index("c")
        sid = lax.axis_index("s")
        wid = sid * NC + cid

        iota16 = lax.iota(jnp.int32, 16)

        d_batch = pltpu.async_copy(batchf_h, batch_v, pre_sem)
        d_ones = pltpu.async_copy(ones_h, ones_v, pre_sem)
        d_pat = pltpu.async_copy(pat_h, pat_v, pre_sem)
        d_hpat = pltpu.async_copy(hpat_h, hpat_v, pre_sem)

        @pl.when(sid == 0)
        def _init():
            pltpu.sync_copy(z34_h, sh_xsum)
            pltpu.sync_copy(z16_h, sh_esum)
            pltpu.sync_copy(z8_h, sh_ncnt)
            pltpu.sync_copy(z8_h, sh_ecnt)

        in_descs = {}

        def start_in(k):
            b = k % NBUF
            c = jnp.minimum(wid + NW * k, NUM_CHUNKS - 1)
            sem = in_sems[b]
            in_descs[k] = [
                pltpu.async_copy(
                    batchf_h.at[pl.ds(c * NODE_CHUNK, NODE_CHUNK)],
                    bidx.at[b], sem),
                pltpu.async_copy(xt_h.at[c], xtile.at[b], sem),
                pltpu.async_copy(ei4_h.at[0, c], rowv.at[b], sem),
            ] + [
                pltpu.async_copy(
                    eav_h.at[fb, c * EB + eb],
                    enat2.at[b, pl.ds(fb * 8, 8), pl.ds(eb * 128, 128)],
                    sem)
                for fb in range(2) for eb in range(EB)
            ]

        sc_descs = {}

        def run_chunk(k):
            b = k % NBUF
            for d in in_descs.pop(k):
                d.wait()

            for j in range(EDGE_CHUNK // 16):
                sl = pl.ds((j % (SUB // 16)) * 16, 16)
                g16 = plsc.load_gather(batch_v, [rowv[b, j // (SUB // 16), sl]])
                gidx[b, j // (SUB // 16), sl] = g16

            def dt_body(it, _):
                for gg in range(8):
                    ce0 = (it * 8 + gg) * 16
                    ipos = iota16 + ce0
                    for r in range(16):
                        fr = (iota16 + r) & 15
                        v = plsc.load_gather(enat2.at[b], [fr, ipos])
                        plsc.store_scatter(edata.at[b], [ipos, fr], v)
                return 0

            lax.fori_loop(0, EDGE_CHUNK // 128, dt_body, 0)

            for i in range(XROWS // 16):
                sl = pl.ds(i * 16, 16)
                ng16 = plsc.load_gather(bidx.at[b], [pat_v[sl]])
                sidx[b, sl] = 2 * ng16 + hpat_v[sl]

            valid = (wid + NW * k) < NUM_CHUNKS

            @pl.when(jnp.logical_not(valid))
            def _redirect_to_dummy_row():
                dummy = jnp.full((16,), G, jnp.int32)
                dummy2 = jnp.full((16,), 2 * G, jnp.int32)
                for i in range((NODE_CHUNK + 15) // 16):
                    bidx[b, pl.ds(min(i * 16, NODE_CHUNK - 16), 16)] = dummy
                for i in range(XROWS // 16):
                    sidx[b, pl.ds(i * 16, 16)] = dummy2
                for s in range(EB):
                    for i in range(SUB // 16):
                        gidx[b, s, pl.ds(i * 16, 16)] = dummy

            sem = sc_sems[b]
            ds_ = [
                pltpu.async_copy(xtile.at[b], sh_xsum.at[sidx.at[b]],
                                 sem, add=True),
                pltpu.async_copy(ones_v.at[pl.ds(0, NODE_CHUNK)],
                                 sh_ncnt.at[bidx.at[b]], sem, add=True),
            ]
            for s in range(EB):
                ds_.append(pltpu.async_copy(
                    edata.at[b].at[pl.ds(s * SUB, SUB)],
                    sh_esum.at[gidx.at[b, s]], sem, add=True))
                ds_.append(pltpu.async_copy(
                    ones_v, sh_ecnt.at[gidx.at[b, s]], sem, add=True))
            sc_descs[k] = ds_

        def drain(k):
            for d in sc_descs.pop(k):
                d.wait()

        start_in(0)
        d_batch.wait()
        d_ones.wait()
        d_pat.wait()
        d_hpat.wait()
        plsc.subcore_barrier()

        for k in range(CHUNKS_PER_TILE):
            if k >= 1:
                drain(k - 1)
            if k + 1 < CHUNKS_PER_TILE:
                start_in(k + 1)
            run_chunk(k)
        drain(CHUNKS_PER_TILE - 1)

        plsc.subcore_barrier()

        @pl.when(sid == 0)
        def _writeout():
            pltpu.sync_copy(sh_xsum.at[pl.ds(0, 2 * G)],
                            xsum_h.at[cid].at[pl.ds(0, 2 * G)])
            pltpu.sync_copy(sh_esum.at[pl.ds(0, G)], esum_h.at[cid])
            pltpu.sync_copy(sh_ncnt.at[pl.ds(0, G)], ncnt_h.at[cid])
            pltpu.sync_copy(sh_ecnt.at[pl.ds(0, G)], ecnt_h.at[cid])

    return agg(xt, batch_flat, ei4, eav, ones, pat, hpat, z34, z16, z8)


def _tc_head(xsum2, esum2, ncnt2, ecnt2, u, w1, b1, w2, b2):
    def body(xs_ref, es_ref, nc_ref, ec_ref, u_ref, w1_ref,
             b1_ref, w2_ref, b2_ref, out_ref):
        xsum = xs_ref[0] + xs_ref[1]
        esum = es_ref[0] + es_ref[1]
        ncnt = nc_ref[0] + nc_ref[1]
        ecnt = ec_ref[0] + ec_ref[1]
        xagg = xsum / jnp.maximum(ncnt[:, 0:1], 1.0)
        eagg = esum / jnp.maximum(ecnt[:, 0:1], 1.0)
        h = (jnp.dot(xagg, w1_ref[0:F_X, :],
                     preferred_element_type=jnp.float32)
             + jnp.dot(eagg, w1_ref[F_X:F_X + F_E, :],
                       preferred_element_type=jnp.float32)
             + jnp.dot(u_ref[...], w1_ref[F_X + F_E:, :],
                       preferred_element_type=jnp.float32)
             + b1_ref[...])
        h = jnp.maximum(h, 0.0)
        out_ref[...] = (jnp.dot(h, w2_ref[...],
                                preferred_element_type=jnp.float32)
                        + b2_ref[...])

    return pl.pallas_call(
        body,
        out_shape=jax.ShapeDtypeStruct((G, F_U), jnp.float32),
    )(xsum2, esum2, ncnt2, ecnt2, u, w1, b1, w2, b2)


def kernel(x, edge_index, edge_attr, u, batch, W1, b1, W2, b2):
    xt = (x.reshape(N_NODES // 8, 8, 2, 128)
          .transpose(0, 2, 1, 3)
          .reshape(NUM_CHUNKS, XROWS, 128))
    eav = (edge_attr.T.reshape(2, 8, N_EDGES // 128, 128)
           .transpose(0, 2, 1, 3))
    ei4 = edge_index.reshape(2, NUM_CHUNKS, EB, 128)
    ones = jnp.ones((SUB, CNT_W), jnp.float32)
    k_ar = jnp.arange(XROWS, dtype=jnp.int32)
    pat = 8 * (k_ar // 16) + k_ar % 8
    hpat = (k_ar // 8) % 2
    z34 = jnp.zeros((2 * (G + 1), 128), jnp.float32)
    z16 = jnp.zeros((G + 1, F_E), jnp.float32)
    z8 = jnp.zeros((G + 1, CNT_W), jnp.float32)

    xsum2h, esum2, ncnt2, ecnt2 = _sc_aggregate(
        xt, batch, ei4, eav, ones, pat, hpat, z34, z16, z8)

    xsum2 = xsum2h.reshape(NC, G + 1, 2, 128)[:, :G].reshape(NC, G, F_X)

    return _tc_head(xsum2, esum2, ncnt2, ecnt2, u, W1,
                    b1.reshape(1, H), W2, b2.reshape(1, F_U))


# speedup vs baseline: 21.8404x; 1.1822x over previous
import functools

import jax
import jax.numpy as jnp
from jax import lax
from jax.experimental import pallas as pl
from jax.experimental.pallas import tpu as pltpu
from jax.experimental.pallas import tpu_sc as plsc

G = 16
N_NODES = 10000
N_EDGES = 160000
F_X = 256
F_E = 16
F_U = 256
H = 512

NODE_CHUNK = 40
EDGE_CHUNK = 640
NUM_CHUNKS = 250
EB = EDGE_CHUNK // 128
XROWS = 2 * NODE_CHUNK
SUB = 128
CNT_W = 8
NBUF = 2

NC = 2
NS = 16
NW = NC * NS
CHUNKS_PER_TILE = (NUM_CHUNKS + NW - 1) // NW


def _sc_aggregate(xt, batch_flat, ei4, eav, ones, pat, hpat, z34, z16, z8):
    mesh = plsc.VectorSubcoreMesh(core_axis_name="c", subcore_axis_name="s")

    @functools.partial(
        pl.kernel,
        out_type=(
            jax.ShapeDtypeStruct((NC, 2 * (G + 1), 128), jnp.float32),
            jax.ShapeDtypeStruct((NC, G, F_E), jnp.float32),
            jax.ShapeDtypeStruct((NC, G, CNT_W), jnp.float32),
            jax.ShapeDtypeStruct((NC, G, CNT_W), jnp.float32),
        ),
        mesh=mesh,
        compiler_params=pltpu.CompilerParams(
            needs_layout_passes=False, use_tc_tiling_on_sc=False),
        scratch_types=[
            pltpu.VMEM((N_NODES,), jnp.int32),
            pltpu.VMEM((NBUF, XROWS, 128), jnp.float32),
            pltpu.VMEM((NBUF, NODE_CHUNK), jnp.int32),
            pltpu.VMEM((NBUF, XROWS), jnp.int32),
            pltpu.VMEM((XROWS,), jnp.int32),
            pltpu.VMEM((XROWS,), jnp.int32),
            pltpu.VMEM((NBUF, 16, EDGE_CHUNK), jnp.float32),
            pltpu.VMEM((NBUF, EDGE_CHUNK, F_E), jnp.float32),
            pltpu.VMEM((NBUF, EB, 128), jnp.int32),
            pltpu.VMEM((NBUF, EB, 128), jnp.int32),
            pltpu.VMEM((SUB, CNT_W), jnp.float32),
            pltpu.VMEM_SHARED((2 * (G + 1), 128), jnp.float32),
            pltpu.VMEM_SHARED((G + 1, F_E), jnp.float32),
            pltpu.VMEM_SHARED((G + 1, CNT_W), jnp.float32),
            pltpu.VMEM_SHARED((G + 1, CNT_W), jnp.float32),
            pltpu.SemaphoreType.DMA,
            pltpu.SemaphoreType.DMA,
            pltpu.SemaphoreType.DMA,
            pltpu.SemaphoreType.DMA,
            pltpu.SemaphoreType.DMA,
        ],
    )
    def agg(xt_h, batchf_h, ei4_h, eav_h, ones_h, pat_h, hpat_h,
            z34_h, z16_h, z8_h,
            xsum_h, esum_h, ncnt_h, ecnt_h,
            batch_v, xtile, bidx, sidx, pat_v, hpat_v,
            enat2, edata, rowv, gidx, ones_v,
            sh_xsum, sh_esum, sh_ncnt, sh_ecnt,
            pre_sem, in_sem0, in_sem1, sc_sem0, sc_sem1):
        in_sems = [in_sem0, in_sem1]
        sc_sems = [sc_sem0, sc_sem1]
        cid = lax.axis_index("c")
        sid = lax.axis_index("s")
        wid = sid * NC + cid

        iota16 = lax.iota(jnp.int32, 16)

        d_batch = pltpu.async_copy(batchf_h, batch_v, pre_sem)
        d_ones = pltpu.async_copy(ones_h, ones_v, pre_sem)
        d_pat = pltpu.async_copy(pat_h, pat_v, pre_sem)
        d_hpat = pltpu.async_copy(hpat_h, hpat_v, pre_sem)

        @pl.when(sid == 0)
        def _init():
            pltpu.sync_copy(z34_h, sh_xsum)
            pltpu.sync_copy(z16_h, sh_esum)
            pltpu.sync_copy(z8_h, sh_ncnt)
            pltpu.sync_copy(z8_h, sh_ecnt)

        in_descs = {}

        def start_in(k):
            b = k % NBUF
            c = jnp.minimum(wid + NW * k, NUM_CHUNKS - 1)
            sem = in_sems[b]
            in_descs[k] = [
                pltpu.async_copy(
                    batchf_h.at[pl.ds(c * NODE_CHUNK, NODE_CHUNK)],
                    bidx.at[b], sem),
                pltpu.async_copy(xt_h.at[c], xtile.at[b], sem),
                pltpu.async_copy(ei4_h.at[0, c], rowv.at[b], sem),
            ] + [
                pltpu.async_copy(
                    eav_h.at[fb, c * EB + eb],
                    enat2.at[b, pl.ds(fb * 8, 8), pl.ds(eb * 128, 128)],
                    sem)
                for fb in range(2) for eb in range(EB)
            ]

        sc_descs = {}

        def run_chunk(k):
            b = k % NBUF
            for d in in_descs.pop(k):
                d.wait()

            for j in range(EDGE_CHUNK // 16):
                sl = pl.ds((j % (SUB // 16)) * 16, 16)
                g16 = plsc.load_gather(batch_v, [rowv[b, j // (SUB // 16), sl]])
                gidx[b, j // (SUB // 16), sl] = g16

            def dt_body(it, _):
                for gg in range(4):
                    ce0 = (it * 4 + gg) * 16
                    ipos = iota16 + ce0
                    vs = [plsc.load_gather(enat2.at[b], [(iota16 + r) & 15, ipos])
                          for r in range(16)]
                    for r in range(16):
                        plsc.store_scatter(edata.at[b],
                                           [ipos, (iota16 + r) & 15], vs[r])
                return 0

            lax.fori_loop(0, EDGE_CHUNK // 64, dt_body, 0)

            for i in range(XROWS // 16):
                sl = pl.ds(i * 16, 16)
                ng16 = plsc.load_gather(bidx.at[b], [pat_v[sl]])
                sidx[b, sl] = 2 * ng16 + hpat_v[sl]

            valid = (wid + NW * k) < NUM_CHUNKS

            @pl.when(jnp.logical_not(valid))
            def _redirect_to_dummy_row():
                dummy = jnp.full((16,), G, jnp.int32)
                dummy2 = jnp.full((16,), 2 * G, jnp.int32)
                for i in range((NODE_CHUNK + 15) // 16):
                    bidx[b, pl.ds(min(i * 16, NODE_CHUNK - 16), 16)] = dummy
                for i in range(XROWS // 16):
                    sidx[b, pl.ds(i * 16, 16)] = dummy2
                for s in range(EB):
                    for i in range(SUB // 16):
                        gidx[b, s, pl.ds(i * 16, 16)] = dummy

            sem = sc_sems[b]
            ds_ = [
                pltpu.async_copy(xtile.at[b], sh_xsum.at[sidx.at[b]],
                                 sem, add=True),
                pltpu.async_copy(ones_v.at[pl.ds(0, NODE_CHUNK)],
                                 sh_ncnt.at[bidx.at[b]], sem, add=True),
            ]
            for s in range(EB):
                ds_.append(pltpu.async_copy(
                    edata.at[b].at[pl.ds(s * SUB, SUB)],
                    sh_esum.at[gidx.at[b, s]], sem, add=True))
                ds_.append(pltpu.async_copy(
                    ones_v, sh_ecnt.at[gidx.at[b, s]], sem, add=True))
            sc_descs[k] = ds_

        def drain(k):
            for d in sc_descs.pop(k):
                d.wait()

        start_in(0)
        d_batch.wait()
        d_ones.wait()
        d_pat.wait()
        d_hpat.wait()
        plsc.subcore_barrier()

        for k in range(CHUNKS_PER_TILE):
            if k >= 1:
                drain(k - 1)
            if k + 1 < CHUNKS_PER_TILE:
                start_in(k + 1)
            run_chunk(k)
        drain(CHUNKS_PER_TILE - 1)

        plsc.subcore_barrier()

        @pl.when(sid == 0)
        def _writeout():
            pltpu.sync_copy(sh_xsum.at[pl.ds(0, 2 * G)],
                            xsum_h.at[cid].at[pl.ds(0, 2 * G)])
            pltpu.sync_copy(sh_esum.at[pl.ds(0, G)], esum_h.at[cid])
            pltpu.sync_copy(sh_ncnt.at[pl.ds(0, G)], ncnt_h.at[cid])
            pltpu.sync_copy(sh_ecnt.at[pl.ds(0, G)], ecnt_h.at[cid])

    return agg(xt, batch_flat, ei4, eav, ones, pat, hpat, z34, z16, z8)


def _tc_head(xsum2, esum2, ncnt2, ecnt2, u, w1, b1, w2, b2):
    def body(xs_ref, es_ref, nc_ref, ec_ref, u_ref, w1_ref,
             b1_ref, w2_ref, b2_ref, out_ref):
        xsum = xs_ref[0] + xs_ref[1]
        esum = es_ref[0] + es_ref[1]
        ncnt = nc_ref[0] + nc_ref[1]
        ecnt = ec_ref[0] + ec_ref[1]
        xagg = xsum / jnp.maximum(ncnt[:, 0:1], 1.0)
        eagg = esum / jnp.maximum(ecnt[:, 0:1], 1.0)
        h = (jnp.dot(xagg, w1_ref[0:F_X, :],
                     preferred_element_type=jnp.float32)
             + jnp.dot(eagg, w1_ref[F_X:F_X + F_E, :],
                       preferred_element_type=jnp.float32)
             + jnp.dot(u_ref[...], w1_ref[F_X + F_E:, :],
                       preferred_element_type=jnp.float32)
             + b1_ref[...])
        h = jnp.maximum(h, 0.0)
        out_ref[...] = (jnp.dot(h, w2_ref[...],
                                preferred_element_type=jnp.float32)
                        + b2_ref[...])

    return pl.pallas_call(
        body,
        out_shape=jax.ShapeDtypeStruct((G, F_U), jnp.float32),
    )(xsum2, esum2, ncnt2, ecnt2, u, w1, b1, w2, b2)


def kernel(x, edge_index, edge_attr, u, batch, W1, b1, W2, b2):
    xt = (x.reshape(N_NODES // 8, 8, 2, 128)
          .transpose(0, 2, 1, 3)
          .reshape(NUM_CHUNKS, XROWS, 128))
    eav = (edge_attr.T.reshape(2, 8, N_EDGES // 128, 128)
           .transpose(0, 2, 1, 3))
    ei4 = edge_index.reshape(2, NUM_CHUNKS, EB, 128)
    ones = jnp.ones((SUB, CNT_W), jnp.float32)
    k_ar = jnp.arange(XROWS, dtype=jnp.int32)
    pat = 8 * (k_ar // 16) + k_ar % 8
    hpat = (k_ar // 8) % 2
    z34 = jnp.zeros((2 * (G + 1), 128), jnp.float32)
    z16 = jnp.zeros((G + 1, F_E), jnp.float32)
    z8 = jnp.zeros((G + 1, CNT_W), jnp.float32)

    xsum2h, esum2, ncnt2, ecnt2 = _sc_aggregate(
        xt, batch, ei4, eav, ones, pat, hpat, z34, z16, z8)

    xsum2 = xsum2h.reshape(NC, G + 1, 2, 128)[:, :G].reshape(NC, G, F_X)

    return _tc_head(xsum2, esum2, ncnt2, ecnt2, u, W1,
                    b1.reshape(1, H), W2, b2.reshape(1, F_U))
